# baseline (speedup 1.0000x reference)
import dataclasses
import functools

import jax
import jax.numpy as jnp
from jax import lax
from jax.experimental import pallas as pl
from jax.experimental.pallas import tpu as pltpu
from jax.experimental.pallas import tpu_sc as plsc

H = 128
NUM_GRAPHS = 64
NC = 2
NS = 16
NW = NC * NS
CHUNK = 128

_SC_CP = pltpu.CompilerParams(needs_layout_passes=False)

_PPERM = [32 * (m // 32) + 16 * (m % 2) + (m % 32) // 2 for m in range(H)]


def _mm(a, w):
    return lax.dot_general(a.astype(jnp.float32), w,
                           (((1,), (0,)), ((), ())),
                           preferred_element_type=jnp.float32)


def _sc_gather(A, B, src, dst):
    E = src.shape[0]
    n_chunks = E // CHUNK
    main = n_chunks // NW
    extra = n_chunks - main * NW
    assert main >= 4 and main % 2 == 0
    mesh = plsc.VectorSubcoreMesh(core_axis_name="c", subcore_axis_name="s")

    @functools.partial(
        pl.kernel,
        out_type=jax.ShapeDtypeStruct((E // 2, H), jnp.int32),
        mesh=mesh,
        compiler_params=_SC_CP,
        scratch_types=[
            pltpu.VMEM((main * CHUNK,), jnp.int32),
            pltpu.VMEM((main * CHUNK,), jnp.int32),
            pltpu.VMEM((2, CHUNK, H), jnp.float32),
            pltpu.VMEM((2, CHUNK, H), jnp.float32),
            pltpu.VMEM((2, CHUNK // 2, H), jnp.int32),
            pltpu.SemaphoreType.DMA,
            pltpu.SemaphoreType.DMA,
            pltpu.SemaphoreType.DMA,
            pltpu.SemaphoreType.DMA,
        ],
    )
    def run(a_hbm, b_hbm, src_hbm, dst_hbm, p_hbm, idxs, idxd, bufa, bufb,
            pbuf, g0, g1, w0, w1):
        c = lax.axis_index("c")
        s = lax.axis_index("s")
        wid = s * NC + c
        gsem = (g0, g1)
        wsem = (w0, w1)
        base = wid * main * CHUNK
        pltpu.sync_copy(src_hbm.at[pl.ds(base, main * CHUNK)], idxs)
        pltpu.sync_copy(dst_hbm.at[pl.ds(base, main * CHUNK)], idxd)

        def fire(j, b):
            isl = pl.ds(j * CHUNK, CHUNK)
            pltpu.async_copy(a_hbm.at[idxs.at[isl]], bufa.at[b], gsem[b])
            pltpu.async_copy(b_hbm.at[idxd.at[isl]], bufb.at[b], gsem[b])

        def drain_gather(b):
            isl = pl.ds(0, CHUNK)
            pltpu.make_async_copy(a_hbm.at[idxs.at[isl]], bufa.at[b],
                                  gsem[b]).wait()
            pltpu.make_async_copy(b_hbm.at[idxd.at[isl]], bufb.at[b],
                                  gsem[b]).wait()

        def drain_write(b):
            pltpu.make_async_copy(pbuf.at[b], p_hbm.at[pl.ds(0, CHUNK // 2)],
                                  wsem[b]).wait()

        def add_rows(b):
            @pl.loop(0, CHUNK // 2)
            def _(rr):
                for par in range(2):
                    r = rr * 2 + par
                    for k in range(H // 32):
                        s0 = bufa[b, r, pl.ds(32 * k, 16)] \
                            + bufb[b, r, pl.ds(32 * k, 16)]
                        s1 = bufa[b, r, pl.ds(32 * k + 16, 16)] \
                            + bufb[b, r, pl.ds(32 * k + 16, 16)]
                        t0 = plsc.bitcast(s0, jnp.int32) + 0x8000
                        t1 = plsc.bitcast(s1, jnp.int32) + 0x8000
                        w = lax.shift_right_logical(t0, 16) | (t1 & -0x10000)
                        pbuf[b, rr, pl.ds(64 * par + 16 * k, 16)] = w

        def process(j, b):
            drain_gather(b)
            add_rows(b)
            pltpu.async_copy(pbuf.at[b],
                             p_hbm.at[pl.ds(wid * main * (CHUNK // 2)
                                            + j * (CHUNK // 2), CHUNK // 2)],
                             wsem[b])

        fire(0, 0)
        fire(1, 1)
        process(0, 0)

        @pl.loop(0, (main - 2) // 2)
        def _(p):
            for u in range(2):
                j = 1 + p * 2 + u
                b = (1 + u) % 2
                nb = 1 - b
                drain_write(nb)
                fire(j + 1, nb)
                process(j, b)

        process(main - 1, (main - 1) % 2)

        drain_write(0)

        @pl.when(wid < extra)
        def _():
            row = main * NW + wid
            pltpu.sync_copy(src_hbm.at[pl.ds(row * CHUNK, CHUNK)],
                            idxs.at[pl.ds(0, CHUNK)])
            pltpu.sync_copy(dst_hbm.at[pl.ds(row * CHUNK, CHUNK)],
                            idxd.at[pl.ds(0, CHUNK)])
            fire(0, 0)
            drain_gather(0)
            add_rows(0)
            pltpu.sync_copy(pbuf.at[0],
                            p_hbm.at[pl.ds(row * (CHUNK // 2), CHUNK // 2)])

        drain_write(1)

    return run(A, B, src, dst)


def _sc_scatter(e_arr, dst, n_nodes):
    E = dst.shape[0]
    n_chunks = E // CHUNK
    base_per_w = n_chunks // NW
    extra = n_chunks - base_per_w * NW
    zrows = 80
    n_zch = n_nodes // zrows
    z_per_s = n_zch // NS
    z_extra = n_zch - z_per_s * NS
    mesh = plsc.VectorSubcoreMesh(core_axis_name="c", subcore_axis_name="s")

    main = base_per_w
    assert main >= 4 and main % 2 == 0

    @functools.partial(
        pl.kernel,
        out_type=jax.ShapeDtypeStruct((NC, n_nodes, H), jnp.float32),
        mesh=mesh,
        compiler_params=_SC_CP,
        scratch_types=[
            pltpu.VMEM((2, CHUNK), jnp.int32),
            pltpu.VMEM((2, CHUNK, H), jnp.float32),
            pltpu.VMEM((zrows, H), jnp.float32),
            pltpu.VMEM_SHARED((n_nodes, H), jnp.float32),
            pltpu.SemaphoreType.DMA,
            pltpu.SemaphoreType.DMA,
            pltpu.SemaphoreType.DMA,
            pltpu.SemaphoreType.DMA,
        ],
    )
    def run(e_hbm, dst_hbm, out_hbm, idxd, ebuf, zbuf, accum, l0, l1, s0, s1):
        c = lax.axis_index("c")
        s = lax.axis_index("s")
        wid = s * NC + c
        n_myz = z_per_s + jnp.where(s < z_extra, 1, 0)
        lsem = (l0, l1)
        ssem = (s0, s1)
        base = wid * main * CHUNK

        @pl.loop(0, zrows)
        def _(r):
            for g in range(H // 16):
                zbuf[r, pl.ds(g * 16, 16)] = jnp.zeros((16,), jnp.float32)

        @pl.loop(0, n_myz)
        def _(k):
            pltpu.sync_copy(zbuf, accum.at[pl.ds((s + k * NS) * zrows, zrows)])

        plsc.subcore_barrier()

        def load(j, b):
            off = base + j * CHUNK
            pltpu.async_copy(dst_hbm.at[pl.ds(off, CHUNK)], idxd.at[b],
                             lsem[b])
            pltpu.async_copy(e_hbm.at[pl.ds(off, CHUNK)], ebuf.at[b], lsem[b])

        def drain_load(b):
            pltpu.make_async_copy(dst_hbm.at[pl.ds(0, CHUNK)], idxd.at[b],
                                  lsem[b]).wait()
            pltpu.make_async_copy(e_hbm.at[pl.ds(0, CHUNK)], ebuf.at[b],
                                  lsem[b]).wait()

        def fire_scatter(j, b):
            pltpu.async_copy(ebuf.at[b], accum.at[idxd.at[b]], ssem[b],
                             add=True)

        def drain_scatter(b):
            pltpu.make_async_copy(e_hbm.at[pl.ds(0, CHUNK)],
                                  ebuf.at[b], ssem[b]).wait()

        load(0, 0)
        load(1, 1)
        drain_load(0)
        fire_scatter(0, 0)

        @pl.loop(0, (main - 2) // 2)
        def _(p):
            for u in range(2):
                j = 1 + p * 2 + u
                b = (1 + u) % 2
                nb = 1 - b
                drain_scatter(nb)
                load(j + 1, nb)
                drain_load(b)
                fire_scatter(j, b)

        drain_scatter(0)
        drain_load(1)
        fire_scatter(main - 1, 1)

        @pl.when(wid < extra)
        def _():
            row = main * NW + wid
            off = row * CHUNK
            pltpu.async_copy(dst_hbm.at[pl.ds(off, CHUNK)], idxd.at[0],
                             lsem[0])
            pltpu.async_copy(e_hbm.at[pl.ds(off, CHUNK)], ebuf.at[0],
                             lsem[0])
            drain_load(0)
            pltpu.sync_copy(ebuf.at[0], accum.at[idxd.at[0]], add=True)

        drain_scatter(1)

        plsc.subcore_barrier()

        @pl.loop(0, n_myz)
        def _(k):
            row0 = (s + k * NS) * zrows
            pltpu.sync_copy(accum.at[pl.ds(row0, zrows)],
                            out_hbm.at[c, pl.ds(row0, zrows)])

    return run(e_arr, dst)


_FULL = lambda i: (0, 0)


def _node_embed(x, wemb, bemb, ws1, bs1, wd1):
    n = x.shape[0]
    bn = 2000
    d_in = x.shape[1]

    def body(x_ref, we_ref, be_ref, ws_ref, bs_ref, wd_ref,
             h_ref, a_ref, b_ref):
        h = _mm(x_ref[...], we_ref[...]) + be_ref[...]
        h_ref[...] = h
        a_ref[...] = _mm(h, ws_ref[...]) + bs_ref[...]
        b_ref[...] = _mm(h, wd_ref[...])

    out_sd = jax.ShapeDtypeStruct((n, H), jnp.float32)
    return pl.pallas_call(
        body,
        grid=(n // bn,),
        in_specs=[
            pl.BlockSpec((bn, d_in), lambda i: (i, 0)),
            pl.BlockSpec((d_in, H), _FULL),
            pl.BlockSpec((1, H), _FULL),
            pl.BlockSpec((H, H), _FULL),
            pl.BlockSpec((1, H), _FULL),
            pl.BlockSpec((H, H), _FULL),
        ],
        out_specs=[pl.BlockSpec((bn, H), lambda i: (i, 0))] * 3,
        out_shape=[out_sd, out_sd, out_sd],
    )(x, wemb, bemb, ws1, bs1, wd1)


def _pick_block(e):
    for be in (8192, 8000, 7808, 6400, 5136, 5120, 4096, 2568, 2560):
        if e % be == 0:
            return be
    raise ValueError(f"no block size for {e}")


def _edge_embed(ea, w, b):
    e = ea.shape[0]
    be = _pick_block(e)
    d_in = ea.shape[1]

    def body(x_ref, w_ref, b_ref, o_ref):
        o_ref[...] = _mm(x_ref[...], w_ref[...]) + b_ref[...]

    return pl.pallas_call(
        body,
        grid=(e // be,),
        in_specs=[
            pl.BlockSpec((be, d_in), lambda i: (i, 0)),
            pl.BlockSpec((d_in, H), _FULL),
            pl.BlockSpec((1, H), _FULL),
        ],
        out_specs=pl.BlockSpec((be, H), lambda i: (i, 0)),
        out_shape=jax.ShapeDtypeStruct((e, H), jnp.float32),
    )(ea, w, b)


def _edge_mlp(P, e_in, we, w2, b2):
    e = P.shape[0]
    be = _pick_block(e)

    def body(p_ref, e_ref, we_ref, w2_ref, b2_ref, o_ref):
        p = p_ref[...].astype(jnp.float32)
        t = jnp.maximum(p + _mm(e_ref[...], we_ref[...]), 0.0)
        o_ref[...] = _mm(t, w2_ref[...]) + b2_ref[...]

    return pl.pallas_call(
        body,
        grid=(e // be,),
        in_specs=[
            pl.BlockSpec((be, H), lambda i: (i, 0)),
            pl.BlockSpec((be, H), lambda i: (i, 0)),
            pl.BlockSpec((H, H), _FULL),
            pl.BlockSpec((H, H), _FULL),
            pl.BlockSpec((1, H), _FULL),
        ],
        out_specs=pl.BlockSpec((be, H), lambda i: (i, 0)),
        out_shape=jax.ShapeDtypeStruct((e, H), jnp.float32),
    )(P, e_in, we, w2, b2)


def _node_mlp(h, aggp0, aggp1, wh, wa, b1, w2, b2, wsn, bsn, wdn):
    n = h.shape[0]
    bn = 2000

    def body(h_ref, a0_ref, a1_ref, a2_ref, a3_ref, wh_ref, wa_ref, b1_ref,
             w2_ref, b2_ref, wsn_ref, bsn_ref, wdn_ref, hn_ref, a_ref, b_ref):
        agg = (a0_ref[0] + a1_ref[0]) + (a2_ref[0] + a3_ref[0])
        t = jnp.maximum(
            _mm(h_ref[...], wh_ref[...]) + _mm(agg, wa_ref[...]) + b1_ref[...],
            0.0)
        hn = _mm(t, w2_ref[...]) + b2_ref[...]
        hn_ref[...] = hn
        a_ref[...] = _mm(hn, wsn_ref[...]) + bsn_ref[...]
        b_ref[...] = _mm(hn, wdn_ref[...])

    out_sd = jax.ShapeDtypeStruct((n, H), jnp.float32)
    return pl.pallas_call(
        body,
        grid=(n // bn,),
        in_specs=[
            pl.BlockSpec((bn, H), lambda i: (i, 0)),
            pl.BlockSpec((1, bn, H), lambda i: (0, i, 0)),
            pl.BlockSpec((1, bn, H), lambda i: (1, i, 0)),
            pl.BlockSpec((1, bn, H), lambda i: (0, i, 0)),
            pl.BlockSpec((1, bn, H), lambda i: (1, i, 0)),
            pl.BlockSpec((H, H), _FULL),
            pl.BlockSpec((H, H), _FULL),
            pl.BlockSpec((1, H), _FULL),
            pl.BlockSpec((H, H), _FULL),
            pl.BlockSpec((1, H), _FULL),
            pl.BlockSpec((H, H), _FULL),
            pl.BlockSpec((1, H), _FULL),
            pl.BlockSpec((H, H), _FULL),
        ],
        out_specs=[pl.BlockSpec((bn, H), lambda i: (i, 0))] * 3,
        out_shape=[out_sd, out_sd, out_sd],
    )(h, aggp0, aggp0, aggp1, aggp1, wh, wa, b1, w2, b2, wsn, bsn, wdn)


def _final_node(h, aggp0, aggp1, wh, wa, b1, wv, c0, batch2d):
    n = h.shape[0]
    bn = 2000

    def body(h_ref, a0_ref, a1_ref, a2_ref, a3_ref, wh_ref, wa_ref, b1_ref,
             wv_ref, c0_ref, bat_ref, o_ref):
        i = pl.program_id(0)
        agg = (a0_ref[0] + a1_ref[0]) + (a2_ref[0] + a3_ref[0])
        t = jnp.maximum(
            _mm(h_ref[...], wh_ref[...]) + _mm(agg, wa_ref[...]) + b1_ref[...],
            0.0)
        energy = jnp.sum(t * wv_ref[...], axis=1, keepdims=True) + c0_ref[0, 0]
        gid = lax.broadcasted_iota(jnp.int32, (bn, H), 1)
        onehot = (bat_ref[...] == gid).astype(jnp.float32)
        partial = jnp.sum(onehot * energy, axis=0, keepdims=True)

        @pl.when(i == 0)
        def _():
            o_ref[...] = jnp.zeros_like(o_ref)

        o_ref[...] += partial

    return pl.pallas_call(
        body,
        grid=(n // bn,),
        in_specs=[
            pl.BlockSpec((bn, H), lambda i: (i, 0)),
            pl.BlockSpec((1, bn, H), lambda i: (0, i, 0)),
            pl.BlockSpec((1, bn, H), lambda i: (1, i, 0)),
            pl.BlockSpec((1, bn, H), lambda i: (0, i, 0)),
            pl.BlockSpec((1, bn, H), lambda i: (1, i, 0)),
            pl.BlockSpec((H, H), _FULL),
            pl.BlockSpec((H, H), _FULL),
            pl.BlockSpec((1, H), _FULL),
            pl.BlockSpec((1, H), _FULL),
            pl.BlockSpec((1, 1), _FULL),
            pl.BlockSpec((bn, 1), lambda i: (i, 0)),
        ],
        out_specs=pl.BlockSpec((1, H), _FULL),
        out_shape=jax.ShapeDtypeStruct((1, H), jnp.float32),
    )(h, aggp0, aggp0, aggp1, aggp1, wh, wa, b1, wv, c0, batch2d)


def kernel(x, edge_attr, edge_index, batch, params):
    n = x.shape[0]
    src = edge_index[0]
    dst = edge_index[1]
    layers = params["layers"]
    n_layers = len(layers)

    def b2d(v):
        return v.reshape(1, -1)

    pp = jnp.array(_PPERM, jnp.int32)
    ws = [lp["bond1"]["w"][:H] for lp in layers]
    wd = [lp["bond1"]["w"][H:2 * H] for lp in layers]
    we = [lp["bond1"]["w"][2 * H:, pp] for lp in layers]
    bb1 = [b2d(lp["bond1"]["b"]) for lp in layers]
    w2p = [lp["bond2"]["w"][pp, :] for lp in layers]
    wh = [lp["atom1"]["w"][:H] for lp in layers]
    wa = [lp["atom1"]["w"][H:] for lp in layers]
    ba1 = [b2d(lp["atom1"]["b"]) for lp in layers]

    E = src.shape[0]
    E0 = ((E // 2 + 8191) // 8192) * 8192
    E1 = E - E0
    assert E1 % CHUNK == 0 and ((E1 // CHUNK) // NW) % 2 == 0
    sp = (src[:E0], src[E0:])
    dp = (dst[:E0], dst[E0:])

    h, A, B = _node_embed(x, params["node_emb"]["w"],
                          b2d(params["node_emb"]["b"]),
                          ws[0], bb1[0], wd[0])
    ee_w = params["edge_emb"]["w"]
    ee_b = b2d(params["edge_emb"]["b"])
    e_h = [_edge_embed(edge_attr[:E0], ee_w, ee_b),
           _edge_embed(edge_attr[E0:], ee_w, ee_b)]

    last = layers[-1]
    wv = (last["atom2"]["w"] @ params["readout"]["w"]).reshape(1, H)
    c0 = (last["atom2"]["b"] @ params["readout"]["w"]
          + params["readout"]["b"]).reshape(1, 1)
    batch2d = batch.reshape(-1, 1)

    def unpack_p(p32):
        return lax.bitcast_convert_type(p32, jnp.bfloat16).reshape(-1, H)

    for l in range(n_layers):
        lp = layers[l]
        bb2 = b2d(lp["bond2"]["b"])
        aggs = []
        for t in range(2):
            P = unpack_p(_sc_gather(A, B, sp[t], dp[t]))
            e_h[t] = _edge_mlp(P, e_h[t], we[l], w2p[l], bb2)
            aggs.append(_sc_scatter(e_h[t], dp[t], n))
        if l + 1 < n_layers:
            h, A, B = _node_mlp(h, aggs[0], aggs[1], wh[l], wa[l], ba1[l],
                                lp["atom2"]["w"], b2d(lp["atom2"]["b"]),
                                ws[l + 1], bb1[l + 1], wd[l + 1])
        else:
            out = _final_node(h, aggs[0], aggs[1], wh[l], wa[l], ba1[l],
                              wv, c0, batch2d)

    return out[0, :NUM_GRAPHS]


# speedup vs baseline: 30.8619x; 30.8619x over previous
import dataclasses
import functools

import jax
import jax.numpy as jnp
from jax import lax
from jax.experimental import pallas as pl
from jax.experimental.pallas import tpu as pltpu
from jax.experimental.pallas import tpu_sc as plsc

H = 128
NUM_GRAPHS = 64
NC = 2
NS = 16
NW = NC * NS
CHUNK = 128

_SC_CP = pltpu.CompilerParams()


def _mm(a, w):
    return lax.dot_general(a.astype(jnp.float32), w,
                           (((1,), (0,)), ((), ())),
                           preferred_element_type=jnp.float32)


def _sc_gather(A, B, src, dst):
    E = src.shape[0]
    n_chunks = E // CHUNK
    main = n_chunks // NW
    extra = n_chunks - main * NW
    assert main >= 4 and main % 2 == 0
    mesh = plsc.VectorSubcoreMesh(core_axis_name="c", subcore_axis_name="s")

    @functools.partial(
        pl.kernel,
        out_type=jax.ShapeDtypeStruct((E, H), jnp.float32),
        mesh=mesh,
        compiler_params=_SC_CP,
        scratch_types=[
            pltpu.VMEM((main * CHUNK,), jnp.int32),
            pltpu.VMEM((main * CHUNK,), jnp.int32),
            pltpu.VMEM((2, CHUNK, H), jnp.float32),
            pltpu.VMEM((2, CHUNK, H), jnp.float32),
            pltpu.SemaphoreType.DMA,
            pltpu.SemaphoreType.DMA,
            pltpu.SemaphoreType.DMA,
            pltpu.SemaphoreType.DMA,
        ],
    )
    def run(a_hbm, b_hbm, src_hbm, dst_hbm, p_hbm, idxs, idxd, bufa, bufb,
            g0, g1, w0, w1):
        c = lax.axis_index("c")
        s = lax.axis_index("s")
        wid = s * NC + c
        gsem = (g0, g1)
        wsem = (w0, w1)
        base = wid * main * CHUNK
        pltpu.sync_copy(src_hbm.at[pl.ds(base, main * CHUNK)], idxs)
        pltpu.sync_copy(dst_hbm.at[pl.ds(base, main * CHUNK)], idxd)

        def fire(j, b):
            isl = pl.ds(j * CHUNK, CHUNK)
            pltpu.async_copy(a_hbm.at[idxs.at[isl]], bufa.at[b], gsem[b])
            pltpu.async_copy(b_hbm.at[idxd.at[isl]], bufb.at[b], gsem[b])

        def drain_gather(b):
            isl = pl.ds(0, CHUNK)
            pltpu.make_async_copy(a_hbm.at[idxs.at[isl]], bufa.at[b],
                                  gsem[b]).wait()
            pltpu.make_async_copy(b_hbm.at[idxd.at[isl]], bufb.at[b],
                                  gsem[b]).wait()

        def drain_write(b):
            pltpu.make_async_copy(bufa.at[b], p_hbm.at[pl.ds(0, CHUNK)],
                                  wsem[b]).wait()

        def add_rows(b):
            @pl.loop(0, CHUNK)
            def _(r):
                for g in range(H // 16):
                    v = bufa[b, r, pl.ds(g * 16, 16)] \
                        + bufb[b, r, pl.ds(g * 16, 16)]
                    bufa[b, r, pl.ds(g * 16, 16)] = v

        def process(j, b):
            drain_gather(b)
            add_rows(b)
            pltpu.async_copy(bufa.at[b],
                             p_hbm.at[pl.ds(base + j * CHUNK, CHUNK)],
                             wsem[b])

        fire(0, 0)
        fire(1, 1)
        process(0, 0)

        @pl.loop(0, (main - 2) // 2)
        def _(p):
            for u in range(2):
                j = 1 + p * 2 + u
                b = (1 + u) % 2
                nb = 1 - b
                drain_write(nb)
                fire(j + 1, nb)
                process(j, b)

        process(main - 1, (main - 1) % 2)

        drain_write(0)

        @pl.when(wid < extra)
        def _():
            row = main * NW + wid
            pltpu.sync_copy(src_hbm.at[pl.ds(row * CHUNK, CHUNK)],
                            idxs.at[pl.ds(0, CHUNK)])
            pltpu.sync_copy(dst_hbm.at[pl.ds(row * CHUNK, CHUNK)],
                            idxd.at[pl.ds(0, CHUNK)])
            fire(0, 0)
            drain_gather(0)
            add_rows(0)
            pltpu.sync_copy(bufa.at[0],
                            p_hbm.at[pl.ds(row * CHUNK, CHUNK)])

        drain_write(1)

    return run(A, B, src, dst)


def _sc_scatter(e_arr, dst, n_nodes):
    E = dst.shape[0]
    n_chunks = E // CHUNK
    base_per_w = n_chunks // NW
    extra = n_chunks - base_per_w * NW
    zrows = 80
    n_zch = n_nodes // zrows
    z_per_s = n_zch // NS
    z_extra = n_zch - z_per_s * NS
    mesh = plsc.VectorSubcoreMesh(core_axis_name="c", subcore_axis_name="s")

    main = base_per_w
    assert main >= 4 and main % 2 == 0

    @functools.partial(
        pl.kernel,
        out_type=jax.ShapeDtypeStruct((NC, n_nodes, H), jnp.float32),
        mesh=mesh,
        compiler_params=_SC_CP,
        scratch_types=[
            pltpu.VMEM((2, CHUNK), jnp.int32),
            pltpu.VMEM((2, CHUNK, H), jnp.float32),
            pltpu.VMEM((zrows, H), jnp.float32),
            pltpu.VMEM_SHARED((n_nodes, H), jnp.float32),
            pltpu.SemaphoreType.DMA,
            pltpu.SemaphoreType.DMA,
            pltpu.SemaphoreType.DMA,
            pltpu.SemaphoreType.DMA,
        ],
    )
    def run(e_hbm, dst_hbm, out_hbm, idxd, ebuf, zbuf, accum, l0, l1, s0, s1):
        c = lax.axis_index("c")
        s = lax.axis_index("s")
        wid = s * NC + c
        n_myz = z_per_s + jnp.where(s < z_extra, 1, 0)
        lsem = (l0, l1)
        ssem = (s0, s1)
        base = wid * main * CHUNK

        @pl.loop(0, zrows)
        def _(r):
            for g in range(H // 16):
                zbuf[r, pl.ds(g * 16, 16)] = jnp.zeros((16,), jnp.float32)

        @pl.loop(0, n_myz)
        def _(k):
            pltpu.sync_copy(zbuf, accum.at[pl.ds((s + k * NS) * zrows, zrows)])

        plsc.subcore_barrier()

        def load(j, b):
            off = base + j * CHUNK
            pltpu.async_copy(dst_hbm.at[pl.ds(off, CHUNK)], idxd.at[b],
                             lsem[b])
            pltpu.async_copy(e_hbm.at[pl.ds(off, CHUNK)], ebuf.at[b], lsem[b])

        def drain_load(b):
            pltpu.make_async_copy(dst_hbm.at[pl.ds(0, CHUNK)], idxd.at[b],
                                  lsem[b]).wait()
            pltpu.make_async_copy(e_hbm.at[pl.ds(0, CHUNK)], ebuf.at[b],
                                  lsem[b]).wait()

        def fire_scatter(j, b):
            pltpu.async_copy(ebuf.at[b], accum.at[idxd.at[b]], ssem[b],
                             add=True)

        def drain_scatter(b):
            pltpu.make_async_copy(e_hbm.at[pl.ds(0, CHUNK)],
                                  ebuf.at[b], ssem[b]).wait()

        load(0, 0)
        load(1, 1)
        drain_load(0)
        fire_scatter(0, 0)

        @pl.loop(0, (main - 2) // 2)
        def _(p):
            for u in range(2):
                j = 1 + p * 2 + u
                b = (1 + u) % 2
                nb = 1 - b
                drain_scatter(nb)
                load(j + 1, nb)
                drain_load(b)
                fire_scatter(j, b)

        drain_scatter(0)
        drain_load(1)
        fire_scatter(main - 1, 1)

        @pl.when(wid < extra)
        def _():
            row = main * NW + wid
            off = row * CHUNK
            pltpu.async_copy(dst_hbm.at[pl.ds(off, CHUNK)], idxd.at[0],
                             lsem[0])
            pltpu.async_copy(e_hbm.at[pl.ds(off, CHUNK)], ebuf.at[0],
                             lsem[0])
            drain_load(0)
            pltpu.sync_copy(ebuf.at[0], accum.at[idxd.at[0]], add=True)

        drain_scatter(1)

        plsc.subcore_barrier()

        @pl.loop(0, n_myz)
        def _(k):
            row0 = (s + k * NS) * zrows
            pltpu.sync_copy(accum.at[pl.ds(row0, zrows)],
                            out_hbm.at[c, pl.ds(row0, zrows)])

    return run(e_arr, dst)


_FULL = lambda i: (0, 0)


def _node_embed(x, wemb, bemb, ws1, bs1, wd1):
    n = x.shape[0]
    bn = 2000
    d_in = x.shape[1]

    def body(x_ref, we_ref, be_ref, ws_ref, bs_ref, wd_ref,
             h_ref, a_ref, b_ref):
        h = _mm(x_ref[...], we_ref[...]) + be_ref[...]
        h_ref[...] = h
        a_ref[...] = _mm(h, ws_ref[...]) + bs_ref[...]
        b_ref[...] = _mm(h, wd_ref[...])

    out_sd = jax.ShapeDtypeStruct((n, H), jnp.float32)
    return pl.pallas_call(
        body,
        grid=(n // bn,),
        in_specs=[
            pl.BlockSpec((bn, d_in), lambda i: (i, 0)),
            pl.BlockSpec((d_in, H), _FULL),
            pl.BlockSpec((1, H), _FULL),
            pl.BlockSpec((H, H), _FULL),
            pl.BlockSpec((1, H), _FULL),
            pl.BlockSpec((H, H), _FULL),
        ],
        out_specs=[pl.BlockSpec((bn, H), lambda i: (i, 0))] * 3,
        out_shape=[out_sd, out_sd, out_sd],
    )(x, wemb, bemb, ws1, bs1, wd1)


def _pick_block(e):
    for be in (8192, 8000, 7808, 6400, 5136, 5120, 4096, 2568, 2560):
        if e % be == 0:
            return be
    raise ValueError(f"no block size for {e}")


def _edge_embed(ea, w, b):
    e = ea.shape[0]
    be = _pick_block(e)
    d_in = ea.shape[1]

    def body(x_ref, w_ref, b_ref, o_ref):
        o_ref[...] = _mm(x_ref[...], w_ref[...]) + b_ref[...]

    return pl.pallas_call(
        body,
        grid=(e // be,),
        in_specs=[
            pl.BlockSpec((be, d_in), lambda i: (i, 0)),
            pl.BlockSpec((d_in, H), _FULL),
            pl.BlockSpec((1, H), _FULL),
        ],
        out_specs=pl.BlockSpec((be, H), lambda i: (i, 0)),
        out_shape=jax.ShapeDtypeStruct((e, H), jnp.float32),
    )(ea, w, b)


def _edge_mlp(P, e_in, we, w2, b2):
    e = P.shape[0]
    be = _pick_block(e)

    def body(p_ref, e_ref, we_ref, w2_ref, b2_ref, o_ref):
        p = p_ref[...].astype(jnp.float32)
        t = jnp.maximum(p + _mm(e_ref[...], we_ref[...]), 0.0)
        o_ref[...] = _mm(t, w2_ref[...]) + b2_ref[...]

    return pl.pallas_call(
        body,
        grid=(e // be,),
        in_specs=[
            pl.BlockSpec((be, H), lambda i: (i, 0)),
            pl.BlockSpec((be, H), lambda i: (i, 0)),
            pl.BlockSpec((H, H), _FULL),
            pl.BlockSpec((H, H), _FULL),
            pl.BlockSpec((1, H), _FULL),
        ],
        out_specs=pl.BlockSpec((be, H), lambda i: (i, 0)),
        out_shape=jax.ShapeDtypeStruct((e, H), jnp.float32),
    )(P, e_in, we, w2, b2)


def _node_mlp(h, aggp0, aggp1, wh, wa, b1, w2, b2, wsn, bsn, wdn):
    n = h.shape[0]
    bn = 2000

    def body(h_ref, a0_ref, a1_ref, a2_ref, a3_ref, wh_ref, wa_ref, b1_ref,
             w2_ref, b2_ref, wsn_ref, bsn_ref, wdn_ref, hn_ref, a_ref, b_ref):
        agg = (a0_ref[0] + a1_ref[0]) + (a2_ref[0] + a3_ref[0])
        t = jnp.maximum(
            _mm(h_ref[...], wh_ref[...]) + _mm(agg, wa_ref[...]) + b1_ref[...],
            0.0)
        hn = _mm(t, w2_ref[...]) + b2_ref[...]
        hn_ref[...] = hn
        a_ref[...] = _mm(hn, wsn_ref[...]) + bsn_ref[...]
        b_ref[...] = _mm(hn, wdn_ref[...])

    out_sd = jax.ShapeDtypeStruct((n, H), jnp.float32)
    return pl.pallas_call(
        body,
        grid=(n // bn,),
        in_specs=[
            pl.BlockSpec((bn, H), lambda i: (i, 0)),
            pl.BlockSpec((1, bn, H), lambda i: (0, i, 0)),
            pl.BlockSpec((1, bn, H), lambda i: (1, i, 0)),
            pl.BlockSpec((1, bn, H), lambda i: (0, i, 0)),
            pl.BlockSpec((1, bn, H), lambda i: (1, i, 0)),
            pl.BlockSpec((H, H), _FULL),
            pl.BlockSpec((H, H), _FULL),
            pl.BlockSpec((1, H), _FULL),
            pl.BlockSpec((H, H), _FULL),
            pl.BlockSpec((1, H), _FULL),
            pl.BlockSpec((H, H), _FULL),
            pl.BlockSpec((1, H), _FULL),
            pl.BlockSpec((H, H), _FULL),
        ],
        out_specs=[pl.BlockSpec((bn, H), lambda i: (i, 0))] * 3,
        out_shape=[out_sd, out_sd, out_sd],
    )(h, aggp0, aggp0, aggp1, aggp1, wh, wa, b1, w2, b2, wsn, bsn, wdn)


def _final_node(h, aggp0, aggp1, wh, wa, b1, wv, c0, batch2d):
    n = h.shape[0]
    bn = 2000

    def body(h_ref, a0_ref, a1_ref, a2_ref, a3_ref, wh_ref, wa_ref, b1_ref,
             wv_ref, c0_ref, bat_ref, o_ref):
        i = pl.program_id(0)
        agg = (a0_ref[0] + a1_ref[0]) + (a2_ref[0] + a3_ref[0])
        t = jnp.maximum(
            _mm(h_ref[...], wh_ref[...]) + _mm(agg, wa_ref[...]) + b1_ref[...],
            0.0)
        energy = jnp.sum(t * wv_ref[...], axis=1, keepdims=True) + c0_ref[0, 0]
        gid = lax.broadcasted_iota(jnp.int32, (bn, H), 1)
        onehot = (bat_ref[...] == gid).astype(jnp.float32)
        partial = jnp.sum(onehot * energy, axis=0, keepdims=True)

        @pl.when(i == 0)
        def _():
            o_ref[...] = jnp.zeros_like(o_ref)

        o_ref[...] += partial

    return pl.pallas_call(
        body,
        grid=(n // bn,),
        in_specs=[
            pl.BlockSpec((bn, H), lambda i: (i, 0)),
            pl.BlockSpec((1, bn, H), lambda i: (0, i, 0)),
            pl.BlockSpec((1, bn, H), lambda i: (1, i, 0)),
            pl.BlockSpec((1, bn, H), lambda i: (0, i, 0)),
            pl.BlockSpec((1, bn, H), lambda i: (1, i, 0)),
            pl.BlockSpec((H, H), _FULL),
            pl.BlockSpec((H, H), _FULL),
            pl.BlockSpec((1, H), _FULL),
            pl.BlockSpec((1, H), _FULL),
            pl.BlockSpec((1, 1), _FULL),
            pl.BlockSpec((bn, 1), lambda i: (i, 0)),
        ],
        out_specs=pl.BlockSpec((1, H), _FULL),
        out_shape=jax.ShapeDtypeStruct((1, H), jnp.float32),
    )(h, aggp0, aggp0, aggp1, aggp1, wh, wa, b1, wv, c0, batch2d)


def kernel(x, edge_attr, edge_index, batch, params):
    n = x.shape[0]
    src = edge_index[0]
    dst = edge_index[1]
    layers = params["layers"]
    n_layers = len(layers)

    def b2d(v):
        return v.reshape(1, -1)

    ws = [lp["bond1"]["w"][:H] for lp in layers]
    wd = [lp["bond1"]["w"][H:2 * H] for lp in layers]
    we = [lp["bond1"]["w"][2 * H:] for lp in layers]
    bb1 = [b2d(lp["bond1"]["b"]) for lp in layers]
    w2p = [lp["bond2"]["w"] for lp in layers]
    wh = [lp["atom1"]["w"][:H] for lp in layers]
    wa = [lp["atom1"]["w"][H:] for lp in layers]
    ba1 = [b2d(lp["atom1"]["b"]) for lp in layers]

    E = src.shape[0]
    E0 = ((E // 2 + 8191) // 8192) * 8192
    E1 = E - E0
    assert E1 % CHUNK == 0 and ((E1 // CHUNK) // NW) % 2 == 0
    sp = (src[:E0], src[E0:])
    dp = (dst[:E0], dst[E0:])

    h, A, B = _node_embed(x, params["node_emb"]["w"],
                          b2d(params["node_emb"]["b"]),
                          ws[0], bb1[0], wd[0])
    ee_w = params["edge_emb"]["w"]
    ee_b = b2d(params["edge_emb"]["b"])
    e_h = [_edge_embed(edge_attr[:E0], ee_w, ee_b),
           _edge_embed(edge_attr[E0:], ee_w, ee_b)]

    last = layers[-1]
    wv = (last["atom2"]["w"] @ params["readout"]["w"]).reshape(1, H)
    c0 = (last["atom2"]["b"] @ params["readout"]["w"]
          + params["readout"]["b"]).reshape(1, 1)
    batch2d = batch.reshape(-1, 1)

    for l in range(n_layers):
        lp = layers[l]
        bb2 = b2d(lp["bond2"]["b"])
        aggs = []
        for t in range(2):
            P = _sc_gather(A, B, sp[t], dp[t])
            e_h[t] = _edge_mlp(P, e_h[t], we[l], w2p[l], bb2)
            aggs.append(_sc_scatter(e_h[t], dp[t], n))
        if l + 1 < n_layers:
            h, A, B = _node_mlp(h, aggs[0], aggs[1], wh[l], wa[l], ba1[l],
                                lp["atom2"]["w"], b2d(lp["atom2"]["b"]),
                                ws[l + 1], bb1[l + 1], wd[l + 1])
        else:
            out = _final_node(h, aggs[0], aggs[1], wh[l], wa[l], ba1[l],
                              wv, c0, batch2d)

    return out[0, :NUM_GRAPHS]


# speedup vs baseline: 30.9663x; 1.0034x over previous
import dataclasses
import functools

import jax
import jax.numpy as jnp
from jax import lax
from jax.experimental import pallas as pl
from jax.experimental.pallas import tpu as pltpu
from jax.experimental.pallas import tpu_sc as plsc

H = 128
NUM_GRAPHS = 64
NC = 2
NS = 16
NW = NC * NS
CHUNK = 128

_SC_CP = pltpu.CompilerParams()


def _mm(a, w):
    return lax.dot_general(a.astype(jnp.float32), w,
                           (((1,), (0,)), ((), ())),
                           preferred_element_type=jnp.float32)


def _sc_gather(A, B, src, dst):
    E = src.shape[0]
    n_chunks = E // CHUNK
    main = n_chunks // NW
    extra = n_chunks - main * NW
    assert main >= 4 and main % 2 == 0
    mesh = plsc.VectorSubcoreMesh(core_axis_name="c", subcore_axis_name="s")

    @functools.partial(
        pl.kernel,
        out_type=jax.ShapeDtypeStruct((E, H), jnp.float32),
        mesh=mesh,
        compiler_params=_SC_CP,
        scratch_types=[
            pltpu.VMEM((main * CHUNK,), jnp.int32),
            pltpu.VMEM((main * CHUNK,), jnp.int32),
            pltpu.VMEM((2, CHUNK, H), jnp.float32),
            pltpu.VMEM((2, CHUNK, H), jnp.float32),
            pltpu.VMEM((2, CHUNK, H), jnp.float32),
            pltpu.SemaphoreType.DMA,
            pltpu.SemaphoreType.DMA,
            pltpu.SemaphoreType.DMA,
            pltpu.SemaphoreType.DMA,
        ],
    )
    def run(a_hbm, b_hbm, src_hbm, dst_hbm, p_hbm, idxs, idxd, bufa, bufb,
            obuf, g0, g1, w0, w1):
        c = lax.axis_index("c")
        s = lax.axis_index("s")
        wid = s * NC + c
        gsem = (g0, g1)
        wsem = (w0, w1)
        base = wid * main * CHUNK
        pltpu.sync_copy(src_hbm.at[pl.ds(base, main * CHUNK)], idxs)
        pltpu.sync_copy(dst_hbm.at[pl.ds(base, main * CHUNK)], idxd)

        def fire(j, b):
            isl = pl.ds(j * CHUNK, CHUNK)
            pltpu.async_copy(a_hbm.at[idxs.at[isl]], bufa.at[b], gsem[b])
            pltpu.async_copy(b_hbm.at[idxd.at[isl]], bufb.at[b], gsem[b])

        def drain_gather(b):
            isl = pl.ds(0, CHUNK)
            pltpu.make_async_copy(a_hbm.at[idxs.at[isl]], bufa.at[b],
                                  gsem[b]).wait()
            pltpu.make_async_copy(b_hbm.at[idxd.at[isl]], bufb.at[b],
                                  gsem[b]).wait()

        def drain_write(b):
            pltpu.make_async_copy(obuf.at[b], p_hbm.at[pl.ds(0, CHUNK)],
                                  wsem[b]).wait()

        def add_rows(b):
            @pl.loop(0, CHUNK)
            def _(r):
                for g in range(H // 16):
                    v = bufa[b, r, pl.ds(g * 16, 16)] \
                        + bufb[b, r, pl.ds(g * 16, 16)]
                    obuf[b, r, pl.ds(g * 16, 16)] = v

        def fire_write(j, b):
            pltpu.async_copy(obuf.at[b],
                             p_hbm.at[pl.ds(base + j * CHUNK, CHUNK)],
                             wsem[b])

        def body(j, b, drain):
            fire(j + 1, 1 - b)
            drain_gather(b)
            if drain:
                drain_write(b)
            add_rows(b)
            fire_write(j, b)

        fire(0, 0)
        body(0, 0, False)
        body(1, 1, False)
        body(2, 0, True)

        @pl.loop(0, (main - 4) // 2)
        def _(p):
            for u in range(2):
                j = 3 + p * 2 + u
                body(j, (3 + u) % 2, True)

        drain_gather(1)
        drain_write(1)
        add_rows(1)
        fire_write(main - 1, 1)

        drain_write(0)

        @pl.when(wid < extra)
        def _():
            row = main * NW + wid
            pltpu.sync_copy(src_hbm.at[pl.ds(row * CHUNK, CHUNK)],
                            idxs.at[pl.ds(0, CHUNK)])
            pltpu.sync_copy(dst_hbm.at[pl.ds(row * CHUNK, CHUNK)],
                            idxd.at[pl.ds(0, CHUNK)])
            fire(0, 0)
            drain_gather(0)
            add_rows(0)
            pltpu.sync_copy(obuf.at[0],
                            p_hbm.at[pl.ds(row * CHUNK, CHUNK)])

        drain_write(1)

    return run(A, B, src, dst)


def _sc_scatter(e_arr, dst, n_nodes):
    E = dst.shape[0]
    n_chunks = E // CHUNK
    base_per_w = n_chunks // NW
    extra = n_chunks - base_per_w * NW
    zrows = 80
    n_zch = n_nodes // zrows
    z_per_s = n_zch // NS
    z_extra = n_zch - z_per_s * NS
    mesh = plsc.VectorSubcoreMesh(core_axis_name="c", subcore_axis_name="s")

    main = base_per_w
    assert main >= 4 and main % 2 == 0

    @functools.partial(
        pl.kernel,
        out_type=jax.ShapeDtypeStruct((NC, n_nodes, H), jnp.float32),
        mesh=mesh,
        compiler_params=_SC_CP,
        scratch_types=[
            pltpu.VMEM((2, CHUNK), jnp.int32),
            pltpu.VMEM((2, CHUNK, H), jnp.float32),
            pltpu.VMEM((zrows, H), jnp.float32),
            pltpu.VMEM_SHARED((n_nodes, H), jnp.float32),
            pltpu.SemaphoreType.DMA,
            pltpu.SemaphoreType.DMA,
            pltpu.SemaphoreType.DMA,
            pltpu.SemaphoreType.DMA,
        ],
    )
    def run(e_hbm, dst_hbm, out_hbm, idxd, ebuf, zbuf, accum, l0, l1, s0, s1):
        c = lax.axis_index("c")
        s = lax.axis_index("s")
        wid = s * NC + c
        n_myz = z_per_s + jnp.where(s < z_extra, 1, 0)
        lsem = (l0, l1)
        ssem = (s0, s1)
        base = wid * main * CHUNK

        @pl.loop(0, zrows)
        def _(r):
            for g in range(H // 16):
                zbuf[r, pl.ds(g * 16, 16)] = jnp.zeros((16,), jnp.float32)

        @pl.loop(0, n_myz)
        def _(k):
            pltpu.sync_copy(zbuf, accum.at[pl.ds((s + k * NS) * zrows, zrows)])

        plsc.subcore_barrier()

        def load(j, b):
            off = base + j * CHUNK
            pltpu.async_copy(dst_hbm.at[pl.ds(off, CHUNK)], idxd.at[b],
                             lsem[b])
            pltpu.async_copy(e_hbm.at[pl.ds(off, CHUNK)], ebuf.at[b], lsem[b])

        def drain_load(b):
            pltpu.make_async_copy(dst_hbm.at[pl.ds(0, CHUNK)], idxd.at[b],
                                  lsem[b]).wait()
            pltpu.make_async_copy(e_hbm.at[pl.ds(0, CHUNK)], ebuf.at[b],
                                  lsem[b]).wait()

        def fire_scatter(j, b):
            pltpu.async_copy(ebuf.at[b], accum.at[idxd.at[b]], ssem[b],
                             add=True)

        def drain_scatter(b):
            pltpu.make_async_copy(e_hbm.at[pl.ds(0, CHUNK)],
                                  ebuf.at[b], ssem[b]).wait()

        load(0, 0)
        load(1, 1)
        drain_load(0)
        fire_scatter(0, 0)

        @pl.loop(0, (main - 2) // 2)
        def _(p):
            for u in range(2):
                j = 1 + p * 2 + u
                b = (1 + u) % 2
                nb = 1 - b
                drain_scatter(nb)
                load(j + 1, nb)
                drain_load(b)
                fire_scatter(j, b)

        drain_scatter(0)
        drain_load(1)
        fire_scatter(main - 1, 1)

        @pl.when(wid < extra)
        def _():
            row = main * NW + wid
            off = row * CHUNK
            pltpu.async_copy(dst_hbm.at[pl.ds(off, CHUNK)], idxd.at[0],
                             lsem[0])
            pltpu.async_copy(e_hbm.at[pl.ds(off, CHUNK)], ebuf.at[0],
                             lsem[0])
            drain_load(0)
            pltpu.sync_copy(ebuf.at[0], accum.at[idxd.at[0]], add=True)

        drain_scatter(1)

        plsc.subcore_barrier()

        @pl.loop(0, n_myz)
        def _(k):
            row0 = (s + k * NS) * zrows
            pltpu.sync_copy(accum.at[pl.ds(row0, zrows)],
                            out_hbm.at[c, pl.ds(row0, zrows)])

    return run(e_arr, dst)


_FULL = lambda i: (0, 0)


def _node_embed(x, wemb, bemb, ws1, bs1, wd1):
    n = x.shape[0]
    bn = 2000
    d_in = x.shape[1]

    def body(x_ref, we_ref, be_ref, ws_ref, bs_ref, wd_ref,
             h_ref, a_ref, b_ref):
        h = _mm(x_ref[...], we_ref[...]) + be_ref[...]
        h_ref[...] = h
        a_ref[...] = _mm(h, ws_ref[...]) + bs_ref[...]
        b_ref[...] = _mm(h, wd_ref[...])

    out_sd = jax.ShapeDtypeStruct((n, H), jnp.float32)
    return pl.pallas_call(
        body,
        grid=(n // bn,),
        in_specs=[
            pl.BlockSpec((bn, d_in), lambda i: (i, 0)),
            pl.BlockSpec((d_in, H), _FULL),
            pl.BlockSpec((1, H), _FULL),
            pl.BlockSpec((H, H), _FULL),
            pl.BlockSpec((1, H), _FULL),
            pl.BlockSpec((H, H), _FULL),
        ],
        out_specs=[pl.BlockSpec((bn, H), lambda i: (i, 0))] * 3,
        out_shape=[out_sd, out_sd, out_sd],
    )(x, wemb, bemb, ws1, bs1, wd1)


def _pick_block(e):
    for be in (8192, 8000, 7808, 6400, 5136, 5120, 4096, 2568, 2560):
        if e % be == 0:
            return be
    raise ValueError(f"no block size for {e}")


def _edge_embed(ea, w, b):
    e = ea.shape[0]
    be = _pick_block(e)
    d_in = ea.shape[1]

    def body(x_ref, w_ref, b_ref, o_ref):
        o_ref[...] = _mm(x_ref[...], w_ref[...]) + b_ref[...]

    return pl.pallas_call(
        body,
        grid=(e // be,),
        in_specs=[
            pl.BlockSpec((be, d_in), lambda i: (i, 0)),
            pl.BlockSpec((d_in, H), _FULL),
            pl.BlockSpec((1, H), _FULL),
        ],
        out_specs=pl.BlockSpec((be, H), lambda i: (i, 0)),
        out_shape=jax.ShapeDtypeStruct((e, H), jnp.float32),
    )(ea, w, b)


def _edge_mlp(P, e_in, we, w2, b2):
    e = P.shape[0]
    be = _pick_block(e)

    def body(p_ref, e_ref, we_ref, w2_ref, b2_ref, o_ref):
        p = p_ref[...].astype(jnp.float32)
        t = jnp.maximum(p + _mm(e_ref[...], we_ref[...]), 0.0)
        o_ref[...] = _mm(t, w2_ref[...]) + b2_ref[...]

    return pl.pallas_call(
        body,
        grid=(e // be,),
        in_specs=[
            pl.BlockSpec((be, H), lambda i: (i, 0)),
            pl.BlockSpec((be, H), lambda i: (i, 0)),
            pl.BlockSpec((H, H), _FULL),
            pl.BlockSpec((H, H), _FULL),
            pl.BlockSpec((1, H), _FULL),
        ],
        out_specs=pl.BlockSpec((be, H), lambda i: (i, 0)),
        out_shape=jax.ShapeDtypeStruct((e, H), jnp.float32),
    )(P, e_in, we, w2, b2)


def _node_mlp(h, aggp0, aggp1, wh, wa, b1, w2, b2, wsn, bsn, wdn):
    n = h.shape[0]
    bn = 2000

    def body(h_ref, a0_ref, a1_ref, a2_ref, a3_ref, wh_ref, wa_ref, b1_ref,
             w2_ref, b2_ref, wsn_ref, bsn_ref, wdn_ref, hn_ref, a_ref, b_ref):
        agg = (a0_ref[0] + a1_ref[0]) + (a2_ref[0] + a3_ref[0])
        t = jnp.maximum(
            _mm(h_ref[...], wh_ref[...]) + _mm(agg, wa_ref[...]) + b1_ref[...],
            0.0)
        hn = _mm(t, w2_ref[...]) + b2_ref[...]
        hn_ref[...] = hn
        a_ref[...] = _mm(hn, wsn_ref[...]) + bsn_ref[...]
        b_ref[...] = _mm(hn, wdn_ref[...])

    out_sd = jax.ShapeDtypeStruct((n, H), jnp.float32)
    return pl.pallas_call(
        body,
        grid=(n // bn,),
        in_specs=[
            pl.BlockSpec((bn, H), lambda i: (i, 0)),
            pl.BlockSpec((1, bn, H), lambda i: (0, i, 0)),
            pl.BlockSpec((1, bn, H), lambda i: (1, i, 0)),
            pl.BlockSpec((1, bn, H), lambda i: (0, i, 0)),
            pl.BlockSpec((1, bn, H), lambda i: (1, i, 0)),
            pl.BlockSpec((H, H), _FULL),
            pl.BlockSpec((H, H), _FULL),
            pl.BlockSpec((1, H), _FULL),
            pl.BlockSpec((H, H), _FULL),
            pl.BlockSpec((1, H), _FULL),
            pl.BlockSpec((H, H), _FULL),
            pl.BlockSpec((1, H), _FULL),
            pl.BlockSpec((H, H), _FULL),
        ],
        out_specs=[pl.BlockSpec((bn, H), lambda i: (i, 0))] * 3,
        out_shape=[out_sd, out_sd, out_sd],
    )(h, aggp0, aggp0, aggp1, aggp1, wh, wa, b1, w2, b2, wsn, bsn, wdn)


def _final_node(h, aggp0, aggp1, wh, wa, b1, wv, c0, batch2d):
    n = h.shape[0]
    bn = 2000

    def body(h_ref, a0_ref, a1_ref, a2_ref, a3_ref, wh_ref, wa_ref, b1_ref,
             wv_ref, c0_ref, bat_ref, o_ref):
        i = pl.program_id(0)
        agg = (a0_ref[0] + a1_ref[0]) + (a2_ref[0] + a3_ref[0])
        t = jnp.maximum(
            _mm(h_ref[...], wh_ref[...]) + _mm(agg, wa_ref[...]) + b1_ref[...],
            0.0)
        energy = jnp.sum(t * wv_ref[...], axis=1, keepdims=True) + c0_ref[0, 0]
        gid = lax.broadcasted_iota(jnp.int32, (bn, H), 1)
        onehot = (bat_ref[...] == gid).astype(jnp.float32)
        partial = jnp.sum(onehot * energy, axis=0, keepdims=True)

        @pl.when(i == 0)
        def _():
            o_ref[...] = jnp.zeros_like(o_ref)

        o_ref[...] += partial

    return pl.pallas_call(
        body,
        grid=(n // bn,),
        in_specs=[
            pl.BlockSpec((bn, H), lambda i: (i, 0)),
            pl.BlockSpec((1, bn, H), lambda i: (0, i, 0)),
            pl.BlockSpec((1, bn, H), lambda i: (1, i, 0)),
            pl.BlockSpec((1, bn, H), lambda i: (0, i, 0)),
            pl.BlockSpec((1, bn, H), lambda i: (1, i, 0)),
            pl.BlockSpec((H, H), _FULL),
            pl.BlockSpec((H, H), _FULL),
            pl.BlockSpec((1, H), _FULL),
            pl.BlockSpec((1, H), _FULL),
            pl.BlockSpec((1, 1), _FULL),
            pl.BlockSpec((bn, 1), lambda i: (i, 0)),
        ],
        out_specs=pl.BlockSpec((1, H), _FULL),
        out_shape=jax.ShapeDtypeStruct((1, H), jnp.float32),
    )(h, aggp0, aggp0, aggp1, aggp1, wh, wa, b1, wv, c0, batch2d)


def kernel(x, edge_attr, edge_index, batch, params):
    n = x.shape[0]
    src = edge_index[0]
    dst = edge_index[1]
    layers = params["layers"]
    n_layers = len(layers)

    def b2d(v):
        return v.reshape(1, -1)

    ws = [lp["bond1"]["w"][:H] for lp in layers]
    wd = [lp["bond1"]["w"][H:2 * H] for lp in layers]
    we = [lp["bond1"]["w"][2 * H:] for lp in layers]
    bb1 = [b2d(lp["bond1"]["b"]) for lp in layers]
    w2p = [lp["bond2"]["w"] for lp in layers]
    wh = [lp["atom1"]["w"][:H] for lp in layers]
    wa = [lp["atom1"]["w"][H:] for lp in layers]
    ba1 = [b2d(lp["atom1"]["b"]) for lp in layers]

    E = src.shape[0]
    E0 = ((E // 2 + 8191) // 8192) * 8192
    E1 = E - E0
    assert E1 % CHUNK == 0 and ((E1 // CHUNK) // NW) % 2 == 0
    sp = (src[:E0], src[E0:])
    dp = (dst[:E0], dst[E0:])

    h, A, B = _node_embed(x, params["node_emb"]["w"],
                          b2d(params["node_emb"]["b"]),
                          ws[0], bb1[0], wd[0])
    ee_w = params["edge_emb"]["w"]
    ee_b = b2d(params["edge_emb"]["b"])
    e_h = [_edge_embed(edge_attr[:E0], ee_w, ee_b),
           _edge_embed(edge_attr[E0:], ee_w, ee_b)]

    last = layers[-1]
    wv = (last["atom2"]["w"] @ params["readout"]["w"]).reshape(1, H)
    c0 = (last["atom2"]["b"] @ params["readout"]["w"]
          + params["readout"]["b"]).reshape(1, 1)
    batch2d = batch.reshape(-1, 1)

    for l in range(n_layers):
        lp = layers[l]
        bb2 = b2d(lp["bond2"]["b"])
        aggs = []
        for t in range(2):
            P = _sc_gather(A, B, sp[t], dp[t])
            e_h[t] = _edge_mlp(P, e_h[t], we[l], w2p[l], bb2)
            aggs.append(_sc_scatter(e_h[t], dp[t], n))
        if l + 1 < n_layers:
            h, A, B = _node_mlp(h, aggs[0], aggs[1], wh[l], wa[l], ba1[l],
                                lp["atom2"]["w"], b2d(lp["atom2"]["b"]),
                                ws[l + 1], bb1[l + 1], wd[l + 1])
        else:
            out = _final_node(h, aggs[0], aggs[1], wh[l], wa[l], ba1[l],
                              wv, c0, batch2d)

    return out[0, :NUM_GRAPHS]
